# baseline (speedup 1.0000x reference)
import math

import jax
import jax.numpy as jnp
from jax import lax
from jax.experimental import pallas as pl
from jax.experimental.pallas import tpu as pltpu
from jax.experimental.pallas import tpu_sc as plsc

P = 2048
FN = 128
H = 64
NE = 131072
NT = 65536
LN_DECAY = math.log(0.9)
INV_SQRT2 = 1.0 / math.sqrt(2.0)

_f32 = jnp.float32


def _gelu(x):
    return 0.5 * x * (1.0 + lax.erf(x * INV_SQRT2))


def _layernorm(x, g, b):
    m = jnp.mean(x, axis=-1, keepdims=True)
    v = jnp.mean((x - m) ** 2, axis=-1, keepdims=True)
    return (x - m) * lax.rsqrt(v + 1e-5) * g + b


def _dot(a, b):
    return jnp.dot(a, b, preferred_element_type=jnp.float32)


def _dotH(a, b):
    return jnp.dot(a, b, preferred_element_type=jnp.float32,
                   precision=lax.Precision.HIGHEST)


def _init_body(px, fx, pW, pb, fW, fb, xp_o, xf_o):
    xp_o[...] = _gelu(_dot(px[...], pW[...]) + pb[...])
    xf_o[...] = _gelu(_dot(fx[...], fW[...]) + fb[...])


def _enc_body(na, ha, W1, b1, W2, b2, Wen, aen, Weh, aeh, le_n_o, le_h_o):
    def enc(attr):
        h = _gelu(_dot(attr, W1[...]) + b1[...])
        h = _dot(h, W2[...]) + b2[...]
        return h * jnp.exp(attr[:, 2:3] * LN_DECAY)

    e_n = enc(na[...])
    e_h = enc(ha[...])
    le_n_o[...] = jnp.concatenate(
        [_dot(e_n, _dot(Wen[L], aen[L])) for L in range(2)], axis=1)
    le_h_o[...] = jnp.concatenate(
        [_dot(e_h, _dot(Weh[L], aeh[L])) for L in range(2)], axis=1)


def _prep_body(xp, xf, Wsn, asn, Wdn, adn, Wsh, ash_, Wdh, adh, le_nc, le_hc,
               xsn_o, xsh_o, ptab_o, ftab_o, M_o):
    xsn = _dot(xp[...], Wsn[...])
    xsn_o[...] = xsn
    xsh = _dot(xf[...], Wsh[...])
    xsh_o[...] = xsh
    ls_n = _dot(xsn, asn[...])
    ld_h = _dot(_dot(xp[...], Wdh[...]), adh[...])
    ptab_o[...] = jnp.concatenate([ls_n, ld_h], axis=1)
    ld_n = _dot(_dot(xf[...], Wdn[...]), adn[...])
    ls_h = _dot(xsh, ash_[...])
    ftab_o[...] = jnp.concatenate([ld_n, ls_h], axis=1)
    raw_n = jnp.max(ls_n) + jnp.max(ld_n) + jnp.max(le_nc[...])
    raw_h = jnp.max(ls_h) + jnp.max(ld_h) + jnp.max(le_hc[...])
    M_n = jnp.where(raw_n > 0, raw_n, 0.2 * raw_n)
    M_h = jnp.where(raw_h > 0, raw_h, 0.2 * raw_h)
    M_o[...] = jnp.concatenate(
        [jnp.zeros((1, 16), _f32) + M_n, jnp.zeros((1, 16), _f32) + M_h],
        axis=0)


def _update_body(ct, ch, cn, xp, xf, xsn, xsh, Wl, bl, Wr, bias_n, bias_h,
                 lnpg, lnpb, lnfg, lnfb, ppW, ppb, pfW, pfb, xp_o, xf_o):
    i = pl.program_id(0)
    xpb = xp[pl.ds(i * 256, 256), :]
    ctb = ct[...]
    cnt = jnp.sum(ctb, axis=1, keepdims=True)
    mean = _dotH(ctb, xp[...]) / jnp.maximum(cnt, 1.0)
    sage = _dot(mean, Wl[...]) + bl[...] + _dot(xpb, Wr[...])
    chb = ch[...]
    sh = jnp.sum(chb, axis=1, keepdims=True)
    gat_p = _dotH(chb, xsh[...]) / (sh + 1e-16) + bias_h[...]
    outp = gat_p + sage
    proj = _dot(_layernorm(outp, lnpg[...], lnpb[...]), ppW[...]) + ppb[...]
    xp_o[...] = _gelu(proj + xpb)

    @pl.when(i == 0)
    def _():
        cnb = cn[...]
        sn = jnp.sum(cnb, axis=1, keepdims=True)
        outf = _dotH(cnb, xsn[...]) / (sn + 1e-16) + bias_n[...]
        projf = _dot(_layernorm(outf, lnfg[...], lnfb[...]), pfW[...]) + pfb[...]
        xf_o[...] = _gelu(projf + xf[...])


def _sprep_body(xp, xf, W1p, b1, W1f, a_o, bf_o):
    a_o[...] = _dot(xp[...], W1p[...]) + b1[...]
    bf_o[...] = _dot(xf[...], W1f[...])


def _score_body(a, bf, W2, b2, w3t, b3, o_ref):
    av = a[...]
    bfv = bf[...]
    PB = av.shape[0]
    cols = []
    for c in range(4):
        bslab = bfv[c * 32:(c + 1) * 32, :]
        h1 = _gelu(av[:, None, :] + bslab[None, :, :])
        h2 = _gelu(_dot(h1.reshape(PB * 32, 128), W2[...]) + b2[...])
        h2r = h2.reshape(PB, 32, H)
        cols.append(jnp.sum(h2r * w3t[...], axis=-1) + b3[...])
    o_ref[...] = jnp.concatenate(cols, axis=1)


def _ctemp_body(tsrc, tdst, zeros, ones, ct_o, src_v, dst_v, idx_v, ones_v,
                C_sh):
    cid = lax.axis_index("c")
    sid = lax.axis_index("s")
    pltpu.sync_copy(ones, ones_v)
    base = sid * 4096
    pltpu.sync_copy(tsrc.at[pl.ds(base, 4096)], src_v)
    pltpu.sync_copy(tdst.at[pl.ds(base, 4096)], dst_v)
    for q in range(2):
        qbase = cid * 1024 + q * 512
        pltpu.sync_copy(zeros, C_sh.at[pl.ds(sid * 65536, 65536)])
        plsc.subcore_barrier()

        def step(j, carry):
            off = j * 16
            s16 = src_v[pl.ds(off, 16)]
            d16 = dst_v[pl.ds(off, 16)]
            rel = d16 - qbase
            ok = (rel >= 0) & (rel < 512)
            idx = jnp.where(ok, rel * 2048 + s16, 1048576)
            idx_v[j // 8, pl.ds((j % 8) * 16, 16)] = idx
            return carry

        lax.fori_loop(0, 256, step, 0)

        def scat(k, carry):
            pltpu.sync_copy(ones_v.at[k], C_sh.at[idx_v.at[k]], add=True)
            return carry

        lax.fori_loop(0, 32, scat, 0)
        plsc.subcore_barrier()
        pltpu.sync_copy(
            C_sh.at[pl.ds(sid * 65536, 65536)],
            ct_o.at[pl.ds(qbase * 2048 + sid * 65536, 65536)])
        plsc.subcore_barrier()


def _edge_body(nsrc, ndst, hsrc, hdst, le_n, le_h, ptab, ftab, M, zeros,
               cn_o, ch_o, src_v, dst_v, le_v, pt_v, ft_v, M_v, idx_v, e_v,
               C_sh):
    cid = lax.axis_index("c")
    sid = lax.axis_index("s")
    pltpu.sync_copy(zeros.at[pl.ds(0, 16384)],
                    C_sh.at[pl.ds(sid * 16384, 16384)])
    pltpu.sync_copy(ptab, pt_v)
    pltpu.sync_copy(ftab, ft_v)
    pltpu.sync_copy(M.at[cid], M_v)
    base = sid * 8192

    @pl.when(cid == 0)
    def _():
        pltpu.sync_copy(nsrc.at[pl.ds(base, 8192)], src_v)
        pltpu.sync_copy(ndst.at[pl.ds(base, 8192)], dst_v)
        pltpu.sync_copy(le_n.at[pl.ds(base, 8192)], le_v)

    @pl.when(cid == 1)
    def _():
        pltpu.sync_copy(hsrc.at[pl.ds(base, 8192)], src_v)
        pltpu.sync_copy(hdst.at[pl.ds(base, 8192)], dst_v)
        pltpu.sync_copy(le_h.at[pl.ds(base, 8192)], le_v)

    plsc.subcore_barrier()

    def compute(is_near):
        def step(j, carry):
            off = j * 16
            s16 = src_v[pl.ds(off, 16)]
            d16 = dst_v[pl.ds(off, 16)]
            l16 = le_v[pl.ds(off, 16)]
            if is_near:
                ls = plsc.load_gather(pt_v, [s16 * 2])
                ld = plsc.load_gather(ft_v, [d16 * 2])
                idx = d16 * 2048 + s16
            else:
                ls = plsc.load_gather(ft_v, [s16 * 2 + 1])
                ld = plsc.load_gather(pt_v, [d16 * 2 + 1])
                idx = d16 * 128 + s16
            logit = ls + ld + l16
            logit = jnp.where(logit > 0, logit, logit * 0.2)
            e16 = jnp.exp(logit - M_v[...])
            e_v[j // 8, pl.ds((j % 8) * 16, 16)] = e16
            idx_v[j // 8, pl.ds((j % 8) * 16, 16)] = idx
            return carry

        lax.fori_loop(0, 512, step, 0)

        def scat(k, carry):
            pltpu.sync_copy(e_v.at[k], C_sh.at[idx_v.at[k]], add=True)
            return carry

        lax.fori_loop(0, 64, scat, 0)

    @pl.when(cid == 0)
    def _():
        compute(True)

    @pl.when(cid == 1)
    def _():
        compute(False)

    plsc.subcore_barrier()

    @pl.when(cid == 0)
    def _():
        pltpu.sync_copy(C_sh.at[pl.ds(sid * 16384, 16384)],
                        cn_o.at[pl.ds(sid * 16384, 16384)])

    @pl.when(cid == 1)
    def _():
        pltpu.sync_copy(C_sh.at[pl.ds(sid * 16384, 16384)],
                        ch_o.at[pl.ds(sid * 16384, 16384)])


def _sds(shape):
    return jax.ShapeDtypeStruct(shape, _f32)


def _full_spec(shape):
    rank = len(shape)
    return pl.BlockSpec(shape, lambda i, _r=rank: (0,) * _r)


def _sc_mesh():
    return plsc.VectorSubcoreMesh(core_axis_name="c", subcore_axis_name="s",
                                  num_cores=2, num_subcores=16)


def _build_ctemp(tsrc, tdst, zeros64k, ones):
    i32 = jnp.int32
    call = pl.kernel(
        _ctemp_body,
        out_type=_sds((P * P,)),
        mesh=_sc_mesh(),
        compiler_params=pltpu.CompilerParams(needs_layout_passes=False),
        scratch_types=[
            pltpu.VMEM((4096,), i32),
            pltpu.VMEM((4096,), i32),
            pltpu.VMEM((32, 128), i32),
            pltpu.VMEM((32, 128), _f32),
            pltpu.VMEM_SHARED((1048592,), _f32),
        ],
    )
    return call(tsrc, tdst, zeros64k, ones)


def _build_edges(nsrc, ndst, hsrc, hdst, le_nc, le_hc, ptab, ftab, M,
                 zeros64k):
    i32 = jnp.int32
    call = pl.kernel(
        _edge_body,
        out_type=(_sds((FN * P,)), _sds((P * FN,))),
        mesh=_sc_mesh(),
        compiler_params=pltpu.CompilerParams(needs_layout_passes=False),
        scratch_types=[
            pltpu.VMEM((8192,), i32),
            pltpu.VMEM((8192,), i32),
            pltpu.VMEM((8192,), _f32),
            pltpu.VMEM((2 * P,), _f32),
            pltpu.VMEM((2 * FN,), _f32),
            pltpu.VMEM((16,), _f32),
            pltpu.VMEM((64, 128), i32),
            pltpu.VMEM((64, 128), _f32),
            pltpu.VMEM_SHARED((P * FN,), _f32),
        ],
    )
    return call(nsrc, ndst, hsrc, hdst, le_nc, le_hc, ptab, ftab, M, zeros64k)


def kernel(plume_x, facility_x, near_attr, hist_attr, params, near_src,
           near_dst, hist_src, hist_dst, temp_src, temp_dst):
    r1 = lambda v: v.reshape(1, -1)
    c1 = lambda v: v.reshape(-1, 1)
    i32 = jnp.int32
    nsrc = near_src.astype(i32)
    ndst = near_dst.astype(i32)
    hsrc = hist_src.astype(i32)
    hdst = hist_dst.astype(i32)
    tsrc = temp_src.astype(i32)
    tdst = temp_dst.astype(i32)

    xp, xf = pl.pallas_call(
        _init_body,
        out_shape=[_sds((P, H)), _sds((FN, H))],
    )(plume_x, facility_x, params['plume_W'], r1(params['plume_b']),
      params['fac_W'], r1(params['fac_b']))

    layers = params['layers']
    Wen = jnp.stack([L['near']['Wedge'] for L in layers])
    aen = jnp.stack([c1(L['near']['aedge']) for L in layers])
    Weh = jnp.stack([L['hist']['Wedge'] for L in layers])
    aeh = jnp.stack([c1(L['hist']['aedge']) for L in layers])

    EB = 8192
    le_near, le_hist = pl.pallas_call(
        _enc_body,
        grid=(NE // EB,),
        in_specs=[
            pl.BlockSpec((EB, 4), lambda i: (i, 0)),
            pl.BlockSpec((EB, 4), lambda i: (i, 0)),
            _full_spec((4, 32)), _full_spec((1, 32)),
            _full_spec((32, 32)), _full_spec((1, 32)),
            _full_spec((2, 32, H)), _full_spec((2, H, 1)),
            _full_spec((2, 32, H)), _full_spec((2, H, 1)),
        ],
        out_specs=[
            pl.BlockSpec((EB, 2), lambda i: (i, 0)),
            pl.BlockSpec((EB, 2), lambda i: (i, 0)),
        ],
        out_shape=[_sds((NE, 2)), _sds((NE, 2))],
    )(near_attr, hist_attr, params['enc_W1'], r1(params['enc_b1']),
      params['enc_W2'], r1(params['enc_b2']), Wen, aen, Weh, aeh)

    zeros64k = jnp.zeros((65536,), _f32)
    ones = jnp.ones((32, 128), _f32)

    ct2d = _build_ctemp(tsrc, tdst, zeros64k, ones).reshape(P, P)

    for L in range(2):
        lw = layers[L]
        ln, lh = lw['near'], lw['hist']
        le_nc = lax.slice(le_near, (0, L), (NE, L + 1)).reshape(-1)
        le_hc = lax.slice(le_hist, (0, L), (NE, L + 1)).reshape(-1)
        xsn, xsh, ptab, ftab, M = pl.pallas_call(
            _prep_body,
            out_shape=[_sds((P, H)), _sds((FN, H)), _sds((P, 2)),
                       _sds((FN, 2)), _sds((2, 16))],
        )(xp, xf, ln['Wsrc'], c1(ln['asrc']), ln['Wdst'], c1(ln['adst']),
          lh['Wsrc'], c1(lh['asrc']), lh['Wdst'], c1(lh['adst']),
          le_nc.reshape(NE // 128, 128), le_hc.reshape(NE // 128, 128))

        cn_flat, ch_flat = _build_edges(
            nsrc, ndst, hsrc, hdst, le_nc, le_hc,
            ptab.reshape(-1), ftab.reshape(-1), M, zeros64k)

        sg = lw['sage']
        xp, xf = pl.pallas_call(
            _update_body,
            grid=(P // 256,),
            in_specs=[
                pl.BlockSpec((256, P), lambda i: (i, 0)),
                pl.BlockSpec((256, FN), lambda i: (i, 0)),
                _full_spec((FN, P)),
                _full_spec((P, H)), _full_spec((FN, H)),
                _full_spec((P, H)), _full_spec((FN, H)),
                _full_spec((H, H)), _full_spec((1, H)), _full_spec((H, H)),
                _full_spec((1, H)), _full_spec((1, H)),
                _full_spec((1, H)), _full_spec((1, H)),
                _full_spec((1, H)), _full_spec((1, H)),
                _full_spec((H, H)), _full_spec((1, H)),
                _full_spec((H, H)), _full_spec((1, H)),
            ],
            out_specs=[
                pl.BlockSpec((256, H), lambda i: (i, 0)),
                pl.BlockSpec((FN, H), lambda i: (0, 0)),
            ],
            out_shape=[_sds((P, H)), _sds((FN, H))],
        )(ct2d, ch_flat.reshape(P, FN), cn_flat.reshape(FN, P), xp, xf,
          xsn, xsh, sg['Wl'], r1(sg['bl']), sg['Wr'], r1(ln['bias']),
          r1(lh['bias']), r1(lw['ln_p_g']), r1(lw['ln_p_b']),
          r1(lw['ln_f_g']), r1(lw['ln_f_b']), lw['proj_p_W'],
          r1(lw['proj_p_b']), lw['proj_f_W'], r1(lw['proj_f_b']))

    a, bf = pl.pallas_call(
        _sprep_body,
        out_shape=[_sds((P, 128)), _sds((FN, 128))],
    )(xp, xf, params['s_W1'][:H], r1(params['s_b1']), params['s_W1'][H:])

    PB = 128
    out = pl.pallas_call(
        _score_body,
        grid=(P // PB,),
        in_specs=[
            pl.BlockSpec((PB, 128), lambda i: (i, 0)),
            _full_spec((FN, 128)),
            _full_spec((128, H)), _full_spec((1, H)),
            _full_spec((1, 1, H)), _full_spec((1, 1)),
        ],
        out_specs=pl.BlockSpec((PB, FN), lambda i: (i, 0)),
        out_shape=_sds((P, FN)),
    )(a, bf, params['s_W2'], r1(params['s_b2']),
      params['s_W3'].reshape(1, 1, H), params['s_b3'].reshape(1, 1))

    return out


# speedup vs baseline: 1.0130x; 1.0130x over previous
import math

import jax
import jax.numpy as jnp
from jax import lax
from jax.experimental import pallas as pl
from jax.experimental.pallas import tpu as pltpu
from jax.experimental.pallas import tpu_sc as plsc

P = 2048
FN = 128
H = 64
NE = 131072
NT = 65536
LN_DECAY = math.log(0.9)
INV_SQRT2 = 1.0 / math.sqrt(2.0)

_f32 = jnp.float32


def _gelu(x):
    return 0.5 * x * (1.0 + lax.erf(x * INV_SQRT2))


def _layernorm(x, g, b):
    m = jnp.mean(x, axis=-1, keepdims=True)
    v = jnp.mean((x - m) ** 2, axis=-1, keepdims=True)
    return (x - m) * lax.rsqrt(v + 1e-5) * g + b


def _dot(a, b):
    return jnp.dot(a, b, preferred_element_type=jnp.float32)


def _dotH(a, b):
    return jnp.dot(a, b, preferred_element_type=jnp.float32,
                   precision=lax.Precision.HIGHEST)


def _init_body(px, fx, pW, pb, fW, fb, xp_o, xf_o):
    xp_o[...] = _gelu(_dot(px[...], pW[...]) + pb[...])
    xf_o[...] = _gelu(_dot(fx[...], fW[...]) + fb[...])


def _enc_body(na, ha, W1, b1, W2, b2, Wen, aen, Weh, aeh, le_n_o, le_h_o):
    def enc(attr):
        h = _gelu(_dot(attr, W1[...]) + b1[...])
        h = _dot(h, W2[...]) + b2[...]
        return h * jnp.exp(attr[:, 2:3] * LN_DECAY)

    e_n = enc(na[...])
    e_h = enc(ha[...])
    le_n_o[...] = jnp.concatenate(
        [_dot(e_n, _dot(Wen[L], aen[L])) for L in range(2)], axis=1)
    le_h_o[...] = jnp.concatenate(
        [_dot(e_h, _dot(Weh[L], aeh[L])) for L in range(2)], axis=1)


def _prep_body(xp, xf, Wsn, asn, Wdn, adn, Wsh, ash_, Wdh, adh, le_nc, le_hc,
               xsn_o, xsh_o, ptab_o, ftab_o, M_o):
    xsn = _dot(xp[...], Wsn[...])
    xsn_o[...] = xsn
    xsh = _dot(xf[...], Wsh[...])
    xsh_o[...] = xsh
    ls_n = _dot(xsn, asn[...])
    ld_h = _dot(_dot(xp[...], Wdh[...]), adh[...])
    ptab_o[...] = jnp.concatenate([ls_n, ld_h], axis=1)
    ld_n = _dot(_dot(xf[...], Wdn[...]), adn[...])
    ls_h = _dot(xsh, ash_[...])
    ftab_o[...] = jnp.concatenate([ld_n, ls_h], axis=1)
    raw_n = jnp.max(ls_n) + jnp.max(ld_n) + jnp.max(le_nc[...])
    raw_h = jnp.max(ls_h) + jnp.max(ld_h) + jnp.max(le_hc[...])
    M_n = jnp.where(raw_n > 0, raw_n, 0.2 * raw_n)
    M_h = jnp.where(raw_h > 0, raw_h, 0.2 * raw_h)
    M_o[...] = jnp.concatenate(
        [jnp.zeros((1, 16), _f32) + M_n, jnp.zeros((1, 16), _f32) + M_h],
        axis=0)


def _update_body(ct, ch, cn, xp, xf, xsn, xsh, Wl, bl, Wr, bias_n, bias_h,
                 lnpg, lnpb, lnfg, lnfb, ppW, ppb, pfW, pfb, xp_o, xf_o):
    i = pl.program_id(0)
    xpb = xp[pl.ds(i * 256, 256), :]
    ctb = ct[...]
    cnt = jnp.sum(ctb, axis=1, keepdims=True)
    mean = _dotH(ctb, xp[...]) / jnp.maximum(cnt, 1.0)
    sage = _dot(mean, Wl[...]) + bl[...] + _dot(xpb, Wr[...])
    chb = ch[...]
    sh = jnp.sum(chb, axis=1, keepdims=True)
    gat_p = _dotH(chb, xsh[...]) / (sh + 1e-16) + bias_h[...]
    outp = gat_p + sage
    proj = _dot(_layernorm(outp, lnpg[...], lnpb[...]), ppW[...]) + ppb[...]
    xp_o[...] = _gelu(proj + xpb)

    @pl.when(i == 0)
    def _():
        cnb = cn[...]
        sn = jnp.sum(cnb, axis=1, keepdims=True)
        outf = _dotH(cnb, xsn[...]) / (sn + 1e-16) + bias_n[...]
        projf = _dot(_layernorm(outf, lnfg[...], lnfb[...]), pfW[...]) + pfb[...]
        xf_o[...] = _gelu(projf + xf[...])


def _sprep_body(xp, xf, W1p, b1, W1f, a_o, bf_o):
    a_o[...] = _dot(xp[...], W1p[...]) + b1[...]
    bf_o[...] = _dot(xf[...], W1f[...])


def _score_body(a, bf, W2, b2, w3t, b3, o_ref):
    av = a[...]
    bfv = bf[...]
    PB = av.shape[0]
    cols = []
    for c in range(4):
        bslab = bfv[c * 32:(c + 1) * 32, :]
        h1 = _gelu(av[:, None, :] + bslab[None, :, :])
        h2 = _gelu(_dot(h1.reshape(PB * 32, 128), W2[...]) + b2[...])
        h2r = h2.reshape(PB, 32, H)
        cols.append(jnp.sum(h2r * w3t[...], axis=-1) + b3[...])
    o_ref[...] = jnp.concatenate(cols, axis=1)


def _ctemp_body(tsrc, tdst, zeros, ones, ct_o, src_v, dst_v, idx_v, ones_v,
                C_sh):
    cid = lax.axis_index("c")
    sid = lax.axis_index("s")
    pltpu.sync_copy(ones, ones_v)
    base = sid * 4096
    pltpu.sync_copy(tsrc.at[pl.ds(base, 4096)], src_v)
    pltpu.sync_copy(tdst.at[pl.ds(base, 4096)], dst_v)
    for q in range(2):
        qbase = cid * 1024 + q * 512
        pltpu.sync_copy(zeros, C_sh.at[pl.ds(sid * 65536, 65536)])
        plsc.subcore_barrier()

        def step(j, carry):
            off = j * 16
            s16 = src_v[pl.ds(off, 16)]
            d16 = dst_v[pl.ds(off, 16)]
            rel = d16 - qbase
            ok = (rel >= 0) & (rel < 512)
            idx = jnp.where(ok, rel * 2048 + s16, 1048576)
            idx_v[j // 8, pl.ds((j % 8) * 16, 16)] = idx
            return carry

        lax.fori_loop(0, 256, step, 0)

        def scat(k, carry):
            pltpu.sync_copy(ones_v.at[k], C_sh.at[idx_v.at[k]], add=True)
            return carry

        lax.fori_loop(0, 32, scat, 0)
        plsc.subcore_barrier()
        pltpu.sync_copy(
            C_sh.at[pl.ds(sid * 65536, 65536)],
            ct_o.at[pl.ds(qbase * 2048 + sid * 65536, 65536)])
        plsc.subcore_barrier()


def _edge_body(nsrc, ndst, hsrc, hdst, le_n, le_h, ptab, ftab, M, zeros,
               cn_o, ch_o, src_v, dst_v, le_v, pt_v, ft_v, M_v, idx_v, e_v,
               C_sh):
    cid = lax.axis_index("c")
    sid = lax.axis_index("s")
    pltpu.sync_copy(zeros.at[pl.ds(0, 16384)],
                    C_sh.at[pl.ds(sid * 16384, 16384)])
    pltpu.sync_copy(ptab, pt_v)
    pltpu.sync_copy(ftab, ft_v)
    pltpu.sync_copy(M.at[cid], M_v)
    base = sid * 8192

    @pl.when(cid == 0)
    def _():
        pltpu.sync_copy(nsrc.at[pl.ds(base, 8192)], src_v)
        pltpu.sync_copy(ndst.at[pl.ds(base, 8192)], dst_v)
        pltpu.sync_copy(le_n.at[pl.ds(base, 8192)], le_v)

    @pl.when(cid == 1)
    def _():
        pltpu.sync_copy(hsrc.at[pl.ds(base, 8192)], src_v)
        pltpu.sync_copy(hdst.at[pl.ds(base, 8192)], dst_v)
        pltpu.sync_copy(le_h.at[pl.ds(base, 8192)], le_v)

    plsc.subcore_barrier()

    def compute(is_near):
        def step(j, carry):
            off = j * 16
            s16 = src_v[pl.ds(off, 16)]
            d16 = dst_v[pl.ds(off, 16)]
            l16 = le_v[pl.ds(off, 16)]
            if is_near:
                ls = plsc.load_gather(pt_v, [s16 * 2])
                ld = plsc.load_gather(ft_v, [d16 * 2])
                idx = d16 * 2048 + s16
            else:
                ls = plsc.load_gather(ft_v, [s16 * 2 + 1])
                ld = plsc.load_gather(pt_v, [d16 * 2 + 1])
                idx = d16 * 128 + s16
            logit = ls + ld + l16
            logit = jnp.where(logit > 0, logit, logit * 0.2)
            e16 = jnp.exp(logit - M_v[...])
            e_v[pl.ds(off, 16)] = e16
            idx_v[pl.ds(off, 16)] = idx
            return carry

        lax.fori_loop(0, 512, step, 0)
        pltpu.sync_copy(e_v, C_sh.at[idx_v], add=True)

    @pl.when(cid == 0)
    def _():
        compute(True)

    @pl.when(cid == 1)
    def _():
        compute(False)

    plsc.subcore_barrier()

    @pl.when(cid == 0)
    def _():
        pltpu.sync_copy(C_sh.at[pl.ds(sid * 16384, 16384)],
                        cn_o.at[pl.ds(sid * 16384, 16384)])

    @pl.when(cid == 1)
    def _():
        pltpu.sync_copy(C_sh.at[pl.ds(sid * 16384, 16384)],
                        ch_o.at[pl.ds(sid * 16384, 16384)])


def _sds(shape):
    return jax.ShapeDtypeStruct(shape, _f32)


def _full_spec(shape):
    rank = len(shape)
    return pl.BlockSpec(shape, lambda i, _r=rank: (0,) * _r)


def _sc_mesh():
    return plsc.VectorSubcoreMesh(core_axis_name="c", subcore_axis_name="s",
                                  num_cores=2, num_subcores=16)


def _build_ctemp(tsrc, tdst, zeros64k, ones):
    i32 = jnp.int32
    call = pl.kernel(
        _ctemp_body,
        out_type=_sds((P * P,)),
        mesh=_sc_mesh(),
        compiler_params=pltpu.CompilerParams(needs_layout_passes=False),
        scratch_types=[
            pltpu.VMEM((4096,), i32),
            pltpu.VMEM((4096,), i32),
            pltpu.VMEM((32, 128), i32),
            pltpu.VMEM((32, 128), _f32),
            pltpu.VMEM_SHARED((1048592,), _f32),
        ],
    )
    return call(tsrc, tdst, zeros64k, ones)


def _build_edges(nsrc, ndst, hsrc, hdst, le_nc, le_hc, ptab, ftab, M,
                 zeros64k):
    i32 = jnp.int32
    call = pl.kernel(
        _edge_body,
        out_type=(_sds((FN * P,)), _sds((P * FN,))),
        mesh=_sc_mesh(),
        compiler_params=pltpu.CompilerParams(needs_layout_passes=False),
        scratch_types=[
            pltpu.VMEM((8192,), i32),
            pltpu.VMEM((8192,), i32),
            pltpu.VMEM((8192,), _f32),
            pltpu.VMEM((2 * P,), _f32),
            pltpu.VMEM((2 * FN,), _f32),
            pltpu.VMEM((16,), _f32),
            pltpu.VMEM((8192,), i32),
            pltpu.VMEM((8192,), _f32),
            pltpu.VMEM_SHARED((P * FN,), _f32),
        ],
    )
    return call(nsrc, ndst, hsrc, hdst, le_nc, le_hc, ptab, ftab, M, zeros64k)


def kernel(plume_x, facility_x, near_attr, hist_attr, params, near_src,
           near_dst, hist_src, hist_dst, temp_src, temp_dst):
    r1 = lambda v: v.reshape(1, -1)
    c1 = lambda v: v.reshape(-1, 1)
    i32 = jnp.int32
    nsrc = near_src.astype(i32)
    ndst = near_dst.astype(i32)
    hsrc = hist_src.astype(i32)
    hdst = hist_dst.astype(i32)
    tsrc = temp_src.astype(i32)
    tdst = temp_dst.astype(i32)

    xp, xf = pl.pallas_call(
        _init_body,
        out_shape=[_sds((P, H)), _sds((FN, H))],
    )(plume_x, facility_x, params['plume_W'], r1(params['plume_b']),
      params['fac_W'], r1(params['fac_b']))

    layers = params['layers']
    Wen = jnp.stack([L['near']['Wedge'] for L in layers])
    aen = jnp.stack([c1(L['near']['aedge']) for L in layers])
    Weh = jnp.stack([L['hist']['Wedge'] for L in layers])
    aeh = jnp.stack([c1(L['hist']['aedge']) for L in layers])

    EB = 8192
    le_near, le_hist = pl.pallas_call(
        _enc_body,
        grid=(NE // EB,),
        in_specs=[
            pl.BlockSpec((EB, 4), lambda i: (i, 0)),
            pl.BlockSpec((EB, 4), lambda i: (i, 0)),
            _full_spec((4, 32)), _full_spec((1, 32)),
            _full_spec((32, 32)), _full_spec((1, 32)),
            _full_spec((2, 32, H)), _full_spec((2, H, 1)),
            _full_spec((2, 32, H)), _full_spec((2, H, 1)),
        ],
        out_specs=[
            pl.BlockSpec((EB, 2), lambda i: (i, 0)),
            pl.BlockSpec((EB, 2), lambda i: (i, 0)),
        ],
        out_shape=[_sds((NE, 2)), _sds((NE, 2))],
    )(near_attr, hist_attr, params['enc_W1'], r1(params['enc_b1']),
      params['enc_W2'], r1(params['enc_b2']), Wen, aen, Weh, aeh)

    zeros64k = jnp.zeros((65536,), _f32)
    ones = jnp.ones((32, 128), _f32)

    ct2d = _build_ctemp(tsrc, tdst, zeros64k, ones).reshape(P, P)

    for L in range(2):
        lw = layers[L]
        ln, lh = lw['near'], lw['hist']
        le_nc = lax.slice(le_near, (0, L), (NE, L + 1)).reshape(-1)
        le_hc = lax.slice(le_hist, (0, L), (NE, L + 1)).reshape(-1)
        xsn, xsh, ptab, ftab, M = pl.pallas_call(
            _prep_body,
            out_shape=[_sds((P, H)), _sds((FN, H)), _sds((P, 2)),
                       _sds((FN, 2)), _sds((2, 16))],
        )(xp, xf, ln['Wsrc'], c1(ln['asrc']), ln['Wdst'], c1(ln['adst']),
          lh['Wsrc'], c1(lh['asrc']), lh['Wdst'], c1(lh['adst']),
          le_nc.reshape(NE // 128, 128), le_hc.reshape(NE // 128, 128))

        cn_flat, ch_flat = _build_edges(
            nsrc, ndst, hsrc, hdst, le_nc, le_hc,
            ptab.reshape(-1), ftab.reshape(-1), M, zeros64k)

        sg = lw['sage']
        xp, xf = pl.pallas_call(
            _update_body,
            grid=(P // 256,),
            in_specs=[
                pl.BlockSpec((256, P), lambda i: (i, 0)),
                pl.BlockSpec((256, FN), lambda i: (i, 0)),
                _full_spec((FN, P)),
                _full_spec((P, H)), _full_spec((FN, H)),
                _full_spec((P, H)), _full_spec((FN, H)),
                _full_spec((H, H)), _full_spec((1, H)), _full_spec((H, H)),
                _full_spec((1, H)), _full_spec((1, H)),
                _full_spec((1, H)), _full_spec((1, H)),
                _full_spec((1, H)), _full_spec((1, H)),
                _full_spec((H, H)), _full_spec((1, H)),
                _full_spec((H, H)), _full_spec((1, H)),
            ],
            out_specs=[
                pl.BlockSpec((256, H), lambda i: (i, 0)),
                pl.BlockSpec((FN, H), lambda i: (0, 0)),
            ],
            out_shape=[_sds((P, H)), _sds((FN, H))],
        )(ct2d, ch_flat.reshape(P, FN), cn_flat.reshape(FN, P), xp, xf,
          xsn, xsh, sg['Wl'], r1(sg['bl']), sg['Wr'], r1(ln['bias']),
          r1(lh['bias']), r1(lw['ln_p_g']), r1(lw['ln_p_b']),
          r1(lw['ln_f_g']), r1(lw['ln_f_b']), lw['proj_p_W'],
          r1(lw['proj_p_b']), lw['proj_f_W'], r1(lw['proj_f_b']))

    a, bf = pl.pallas_call(
        _sprep_body,
        out_shape=[_sds((P, 128)), _sds((FN, 128))],
    )(xp, xf, params['s_W1'][:H], r1(params['s_b1']), params['s_W1'][H:])

    PB = 128
    out = pl.pallas_call(
        _score_body,
        grid=(P // PB,),
        in_specs=[
            pl.BlockSpec((PB, 128), lambda i: (i, 0)),
            _full_spec((FN, 128)),
            _full_spec((128, H)), _full_spec((1, H)),
            _full_spec((1, 1, H)), _full_spec((1, 1)),
        ],
        out_specs=pl.BlockSpec((PB, FN), lambda i: (i, 0)),
        out_shape=_sds((P, FN)),
    )(a, bf, params['s_W2'], r1(params['s_b2']),
      params['s_W3'].reshape(1, 1, H), params['s_b3'].reshape(1, 1))

    return out


# speedup vs baseline: 1.0385x; 1.0251x over previous
import math

import jax
import jax.numpy as jnp
from jax import lax
from jax.experimental import pallas as pl
from jax.experimental.pallas import tpu as pltpu
from jax.experimental.pallas import tpu_sc as plsc

P = 2048
FN = 128
H = 64
NE = 131072
NT = 65536
LN_DECAY = math.log(0.9)
INV_SQRT2 = 1.0 / math.sqrt(2.0)

_f32 = jnp.float32


def _gelu(x):
    return 0.5 * x * (1.0 + lax.erf(x * INV_SQRT2))


def _layernorm(x, g, b):
    m = jnp.mean(x, axis=-1, keepdims=True)
    v = jnp.mean((x - m) ** 2, axis=-1, keepdims=True)
    return (x - m) * lax.rsqrt(v + 1e-5) * g + b


def _dot(a, b):
    return jnp.dot(a, b, preferred_element_type=jnp.float32)


def _dotH(a, b):
    return jnp.dot(a, b, preferred_element_type=jnp.float32,
                   precision=lax.Precision.HIGHEST)


def _init_body(px, fx, pW, pb, fW, fb, xp_o, xf_o):
    xp_o[...] = _gelu(_dot(px[...], pW[...]) + pb[...])
    xf_o[...] = _gelu(_dot(fx[...], fW[...]) + fb[...])


def _enc_body(na, ha, W1, b1, W2, b2, Wen, aen, Weh, aeh, le_n_o, le_h_o):
    def enc(attr):
        h = _gelu(_dot(attr, W1[...]) + b1[...])
        h = _dot(h, W2[...]) + b2[...]
        return h * jnp.exp(attr[:, 2:3] * LN_DECAY)

    e_n = enc(na[...])
    e_h = enc(ha[...])
    le_n_o[...] = jnp.concatenate(
        [_dot(e_n, _dot(Wen[L], aen[L])) for L in range(2)], axis=1)
    le_h_o[...] = jnp.concatenate(
        [_dot(e_h, _dot(Weh[L], aeh[L])) for L in range(2)], axis=1)


def _prep_body(xp, xf, Wsn, asn, Wdn, adn, Wsh, ash_, Wdh, adh, le_nc, le_hc,
               xsn_o, xsh_o, ptab_o, ftab_o, M_o):
    xsn = _dot(xp[...], Wsn[...])
    xsn_o[...] = xsn
    xsh = _dot(xf[...], Wsh[...])
    xsh_o[...] = xsh
    ls_n = _dot(xsn, asn[...])
    ld_h = _dot(_dot(xp[...], Wdh[...]), adh[...])
    ptab_o[...] = jnp.concatenate([ls_n, ld_h], axis=1)
    ld_n = _dot(_dot(xf[...], Wdn[...]), adn[...])
    ls_h = _dot(xsh, ash_[...])
    ftab_o[...] = jnp.concatenate([ld_n, ls_h], axis=1)
    raw_n = jnp.max(ls_n) + jnp.max(ld_n) + jnp.max(le_nc[...])
    raw_h = jnp.max(ls_h) + jnp.max(ld_h) + jnp.max(le_hc[...])
    M_n = jnp.where(raw_n > 0, raw_n, 0.2 * raw_n)
    M_h = jnp.where(raw_h > 0, raw_h, 0.2 * raw_h)
    M_o[...] = jnp.concatenate(
        [jnp.zeros((1, 16), _f32) + M_n, jnp.zeros((1, 16), _f32) + M_h],
        axis=0)


def _update_body(ct, ch, cn, xp, xf, xsn, xsh, Wl, bl, Wr, bias_n, bias_h,
                 lnpg, lnpb, lnfg, lnfb, ppW, ppb, pfW, pfb, xp_o, xf_o):
    i = pl.program_id(0)
    xpb = xp[pl.ds(i * 256, 256), :]
    ctb = ct[...]
    cnt = jnp.sum(ctb, axis=1, keepdims=True)
    mean = _dotH(ctb, xp[...]) / jnp.maximum(cnt, 1.0)
    sage = _dot(mean, Wl[...]) + bl[...] + _dot(xpb, Wr[...])
    chb = ch[...]
    sh = jnp.sum(chb, axis=1, keepdims=True)
    gat_p = _dotH(chb, xsh[...]) / (sh + 1e-16) + bias_h[...]
    outp = gat_p + sage
    proj = _dot(_layernorm(outp, lnpg[...], lnpb[...]), ppW[...]) + ppb[...]
    xp_o[...] = _gelu(proj + xpb)

    @pl.when(i == 0)
    def _():
        cnb = cn[...]
        sn = jnp.sum(cnb, axis=1, keepdims=True)
        outf = _dotH(cnb, xsn[...]) / (sn + 1e-16) + bias_n[...]
        projf = _dot(_layernorm(outf, lnfg[...], lnfb[...]), pfW[...]) + pfb[...]
        xf_o[...] = _gelu(projf + xf[...])


def _sprep_body(xp, xf, W1p, b1, W1f, a_o, bf_o):
    a_o[...] = _dot(xp[...], W1p[...]) + b1[...]
    bf_o[...] = _dot(xf[...], W1f[...])


def _score_body(a, bf, W2, b2, w3t, b3, o_ref):
    av = a[...]
    bfv = bf[...]
    PB = av.shape[0]
    cols = []
    for c in range(4):
        bslab = bfv[c * 32:(c + 1) * 32, :]
        h1 = _gelu(av[:, None, :] + bslab[None, :, :])
        h2 = _gelu(_dot(h1.reshape(PB * 32, 128), W2[...]) + b2[...])
        h2r = h2.reshape(PB, 32, H)
        cols.append(jnp.sum(h2r * w3t[...], axis=-1) + b3[...])
    o_ref[...] = jnp.concatenate(cols, axis=1)


def _ctemp_body(tsrc, tdst, zeros, ones, ct_o, src_v, dst_v, idx_v, ones_v,
                C_sh):
    cid = lax.axis_index("c")
    sid = lax.axis_index("s")
    pltpu.sync_copy(ones, ones_v)
    base = sid * 4096
    pltpu.sync_copy(tsrc.at[pl.ds(base, 4096)], src_v)
    pltpu.sync_copy(tdst.at[pl.ds(base, 4096)], dst_v)
    for q in range(2):
        qbase = cid * 1024 + q * 512
        pltpu.sync_copy(zeros, C_sh.at[pl.ds(sid * 65536, 65536)])
        plsc.subcore_barrier()

        @plsc.parallel_loop(0, 256, step=1, unroll=8)
        def step(j):
            off = j * 16
            s16 = src_v[pl.ds(off, 16)]
            d16 = dst_v[pl.ds(off, 16)]
            rel = d16 - qbase
            ok = (rel >= 0) & (rel < 512)
            idx = jnp.where(ok, rel * 2048 + s16, 1048576)
            idx_v[pl.ds(off, 16)] = idx

        pltpu.sync_copy(ones_v, C_sh.at[idx_v], add=True)
        plsc.subcore_barrier()
        pltpu.sync_copy(
            C_sh.at[pl.ds(sid * 65536, 65536)],
            ct_o.at[pl.ds(qbase * 2048 + sid * 65536, 65536)])
        plsc.subcore_barrier()


def _edge_body(nsrc, ndst, hsrc, hdst, le_n, le_h, ptab, ftab, M, zeros,
               cn_o, ch_o, src_v, dst_v, le_v, pt_v, ft_v, M_v, idx_v, e_v,
               C_sh):
    cid = lax.axis_index("c")
    sid = lax.axis_index("s")
    pltpu.sync_copy(zeros.at[pl.ds(0, 16384)],
                    C_sh.at[pl.ds(sid * 16384, 16384)])
    pltpu.sync_copy(ptab, pt_v)
    pltpu.sync_copy(ftab, ft_v)
    pltpu.sync_copy(M.at[cid], M_v)
    base = sid * 8192

    @pl.when(cid == 0)
    def _():
        pltpu.sync_copy(nsrc.at[pl.ds(base, 8192)], src_v)
        pltpu.sync_copy(ndst.at[pl.ds(base, 8192)], dst_v)
        pltpu.sync_copy(le_n.at[pl.ds(base, 8192)], le_v)

    @pl.when(cid == 1)
    def _():
        pltpu.sync_copy(hsrc.at[pl.ds(base, 8192)], src_v)
        pltpu.sync_copy(hdst.at[pl.ds(base, 8192)], dst_v)
        pltpu.sync_copy(le_h.at[pl.ds(base, 8192)], le_v)

    plsc.subcore_barrier()

    def compute(is_near):
        @plsc.parallel_loop(0, 512, step=1, unroll=8)
        def step(j):
            off = j * 16
            s16 = src_v[pl.ds(off, 16)]
            d16 = dst_v[pl.ds(off, 16)]
            l16 = le_v[pl.ds(off, 16)]
            if is_near:
                ls = plsc.load_gather(pt_v, [s16 * 2])
                ld = plsc.load_gather(ft_v, [d16 * 2])
                idx = d16 * 2048 + s16
            else:
                ls = plsc.load_gather(ft_v, [s16 * 2 + 1])
                ld = plsc.load_gather(pt_v, [d16 * 2 + 1])
                idx = d16 * 128 + s16
            logit = ls + ld + l16
            logit = jnp.where(logit > 0, logit, logit * 0.2)
            e16 = jnp.exp(logit - M_v[...])
            e_v[pl.ds(off, 16)] = e16
            idx_v[pl.ds(off, 16)] = idx

        pltpu.sync_copy(e_v, C_sh.at[idx_v], add=True)

    @pl.when(cid == 0)
    def _():
        compute(True)

    @pl.when(cid == 1)
    def _():
        compute(False)

    plsc.subcore_barrier()

    @pl.when(cid == 0)
    def _():
        pltpu.sync_copy(C_sh.at[pl.ds(sid * 16384, 16384)],
                        cn_o.at[pl.ds(sid * 16384, 16384)])

    @pl.when(cid == 1)
    def _():
        pltpu.sync_copy(C_sh.at[pl.ds(sid * 16384, 16384)],
                        ch_o.at[pl.ds(sid * 16384, 16384)])


def _sds(shape):
    return jax.ShapeDtypeStruct(shape, _f32)


def _full_spec(shape):
    rank = len(shape)
    return pl.BlockSpec(shape, lambda i, _r=rank: (0,) * _r)


def _sc_mesh():
    return plsc.VectorSubcoreMesh(core_axis_name="c", subcore_axis_name="s",
                                  num_cores=2, num_subcores=16)


def _build_ctemp(tsrc, tdst, zeros64k, ones):
    i32 = jnp.int32
    call = pl.kernel(
        _ctemp_body,
        out_type=_sds((P * P,)),
        mesh=_sc_mesh(),
        compiler_params=pltpu.CompilerParams(needs_layout_passes=False),
        scratch_types=[
            pltpu.VMEM((4096,), i32),
            pltpu.VMEM((4096,), i32),
            pltpu.VMEM((4096,), i32),
            pltpu.VMEM((4096,), _f32),
            pltpu.VMEM_SHARED((1048592,), _f32),
        ],
    )
    return call(tsrc, tdst, zeros64k, ones)


def _build_edges(nsrc, ndst, hsrc, hdst, le_nc, le_hc, ptab, ftab, M,
                 zeros64k):
    i32 = jnp.int32
    call = pl.kernel(
        _edge_body,
        out_type=(_sds((FN * P,)), _sds((P * FN,))),
        mesh=_sc_mesh(),
        compiler_params=pltpu.CompilerParams(needs_layout_passes=False),
        scratch_types=[
            pltpu.VMEM((8192,), i32),
            pltpu.VMEM((8192,), i32),
            pltpu.VMEM((8192,), _f32),
            pltpu.VMEM((2 * P,), _f32),
            pltpu.VMEM((2 * FN,), _f32),
            pltpu.VMEM((16,), _f32),
            pltpu.VMEM((8192,), i32),
            pltpu.VMEM((8192,), _f32),
            pltpu.VMEM_SHARED((P * FN,), _f32),
        ],
    )
    return call(nsrc, ndst, hsrc, hdst, le_nc, le_hc, ptab, ftab, M, zeros64k)


def kernel(plume_x, facility_x, near_attr, hist_attr, params, near_src,
           near_dst, hist_src, hist_dst, temp_src, temp_dst):
    r1 = lambda v: v.reshape(1, -1)
    c1 = lambda v: v.reshape(-1, 1)
    i32 = jnp.int32
    nsrc = near_src.astype(i32)
    ndst = near_dst.astype(i32)
    hsrc = hist_src.astype(i32)
    hdst = hist_dst.astype(i32)
    tsrc = temp_src.astype(i32)
    tdst = temp_dst.astype(i32)

    xp, xf = pl.pallas_call(
        _init_body,
        out_shape=[_sds((P, H)), _sds((FN, H))],
    )(plume_x, facility_x, params['plume_W'], r1(params['plume_b']),
      params['fac_W'], r1(params['fac_b']))

    layers = params['layers']
    Wen = jnp.stack([L['near']['Wedge'] for L in layers])
    aen = jnp.stack([c1(L['near']['aedge']) for L in layers])
    Weh = jnp.stack([L['hist']['Wedge'] for L in layers])
    aeh = jnp.stack([c1(L['hist']['aedge']) for L in layers])

    EB = 8192
    le_near, le_hist = pl.pallas_call(
        _enc_body,
        grid=(NE // EB,),
        in_specs=[
            pl.BlockSpec((EB, 4), lambda i: (i, 0)),
            pl.BlockSpec((EB, 4), lambda i: (i, 0)),
            _full_spec((4, 32)), _full_spec((1, 32)),
            _full_spec((32, 32)), _full_spec((1, 32)),
            _full_spec((2, 32, H)), _full_spec((2, H, 1)),
            _full_spec((2, 32, H)), _full_spec((2, H, 1)),
        ],
        out_specs=[
            pl.BlockSpec((EB, 2), lambda i: (i, 0)),
            pl.BlockSpec((EB, 2), lambda i: (i, 0)),
        ],
        out_shape=[_sds((NE, 2)), _sds((NE, 2))],
    )(near_attr, hist_attr, params['enc_W1'], r1(params['enc_b1']),
      params['enc_W2'], r1(params['enc_b2']), Wen, aen, Weh, aeh)

    zeros64k = jnp.zeros((65536,), _f32)
    ones = jnp.ones((4096,), _f32)

    ct2d = _build_ctemp(tsrc, tdst, zeros64k, ones).reshape(P, P)

    for L in range(2):
        lw = layers[L]
        ln, lh = lw['near'], lw['hist']
        le_nc = lax.slice(le_near, (0, L), (NE, L + 1)).reshape(-1)
        le_hc = lax.slice(le_hist, (0, L), (NE, L + 1)).reshape(-1)
        xsn, xsh, ptab, ftab, M = pl.pallas_call(
            _prep_body,
            out_shape=[_sds((P, H)), _sds((FN, H)), _sds((P, 2)),
                       _sds((FN, 2)), _sds((2, 16))],
        )(xp, xf, ln['Wsrc'], c1(ln['asrc']), ln['Wdst'], c1(ln['adst']),
          lh['Wsrc'], c1(lh['asrc']), lh['Wdst'], c1(lh['adst']),
          le_nc.reshape(NE // 128, 128), le_hc.reshape(NE // 128, 128))

        cn_flat, ch_flat = _build_edges(
            nsrc, ndst, hsrc, hdst, le_nc, le_hc,
            ptab.reshape(-1), ftab.reshape(-1), M, zeros64k)

        sg = lw['sage']
        xp, xf = pl.pallas_call(
            _update_body,
            grid=(P // 256,),
            in_specs=[
                pl.BlockSpec((256, P), lambda i: (i, 0)),
                pl.BlockSpec((256, FN), lambda i: (i, 0)),
                _full_spec((FN, P)),
                _full_spec((P, H)), _full_spec((FN, H)),
                _full_spec((P, H)), _full_spec((FN, H)),
                _full_spec((H, H)), _full_spec((1, H)), _full_spec((H, H)),
                _full_spec((1, H)), _full_spec((1, H)),
                _full_spec((1, H)), _full_spec((1, H)),
                _full_spec((1, H)), _full_spec((1, H)),
                _full_spec((H, H)), _full_spec((1, H)),
                _full_spec((H, H)), _full_spec((1, H)),
            ],
            out_specs=[
                pl.BlockSpec((256, H), lambda i: (i, 0)),
                pl.BlockSpec((FN, H), lambda i: (0, 0)),
            ],
            out_shape=[_sds((P, H)), _sds((FN, H))],
        )(ct2d, ch_flat.reshape(P, FN), cn_flat.reshape(FN, P), xp, xf,
          xsn, xsh, sg['Wl'], r1(sg['bl']), sg['Wr'], r1(ln['bias']),
          r1(lh['bias']), r1(lw['ln_p_g']), r1(lw['ln_p_b']),
          r1(lw['ln_f_g']), r1(lw['ln_f_b']), lw['proj_p_W'],
          r1(lw['proj_p_b']), lw['proj_f_W'], r1(lw['proj_f_b']))

    a, bf = pl.pallas_call(
        _sprep_body,
        out_shape=[_sds((P, 128)), _sds((FN, 128))],
    )(xp, xf, params['s_W1'][:H], r1(params['s_b1']), params['s_W1'][H:])

    PB = 128
    out = pl.pallas_call(
        _score_body,
        grid=(P // PB,),
        in_specs=[
            pl.BlockSpec((PB, 128), lambda i: (i, 0)),
            _full_spec((FN, 128)),
            _full_spec((128, H)), _full_spec((1, H)),
            _full_spec((1, 1, H)), _full_spec((1, 1)),
        ],
        out_specs=pl.BlockSpec((PB, FN), lambda i: (i, 0)),
        out_shape=_sds((P, FN)),
    )(a, bf, params['s_W2'], r1(params['s_b2']),
      params['s_W3'].reshape(1, 1, H), params['s_b3'].reshape(1, 1))

    return out


# speedup vs baseline: 1.2484x; 1.2022x over previous
import math

import jax
import jax.numpy as jnp
from jax import lax
from jax.experimental import pallas as pl
from jax.experimental.pallas import tpu as pltpu
from jax.experimental.pallas import tpu_sc as plsc

P = 2048
FN = 128
H = 64
NE = 131072
NT = 65536
LN_DECAY = math.log(0.9)
INV_SQRT2 = 1.0 / math.sqrt(2.0)

_f32 = jnp.float32


def _gelu(x):
    return 0.5 * x * (1.0 + lax.erf(x * INV_SQRT2))


def _layernorm(x, g, b):
    m = jnp.mean(x, axis=-1, keepdims=True)
    v = jnp.mean((x - m) ** 2, axis=-1, keepdims=True)
    return (x - m) * lax.rsqrt(v + 1e-5) * g + b


def _dot(a, b):
    return jnp.dot(a, b, preferred_element_type=jnp.float32)


def _dotH(a, b):
    return jnp.dot(a, b, preferred_element_type=jnp.float32,
                   precision=lax.Precision.HIGHEST)


def _init_body(px, fx, pW, pb, fW, fb, xp_o, xf_o):
    xp_o[...] = _gelu(_dot(px[...], pW[...]) + pb[...])
    xf_o[...] = _gelu(_dot(fx[...], fW[...]) + fb[...])


def _enc_body(na, ha, W1, b1, W2, b2, Wen, aen, Weh, aeh, le_n_o, le_h_o):
    def enc(attr):
        h = _gelu(_dot(attr, W1[...]) + b1[...])
        h = _dot(h, W2[...]) + b2[...]
        return h * jnp.exp(attr[:, 2:3] * LN_DECAY)

    e_n = enc(na[...])
    e_h = enc(ha[...])
    le_n_o[...] = jnp.concatenate(
        [_dot(e_n, _dot(Wen[L], aen[L])) for L in range(2)], axis=1)
    le_h_o[...] = jnp.concatenate(
        [_dot(e_h, _dot(Weh[L], aeh[L])) for L in range(2)], axis=1)


def _prep_body(xp, xf, Wsn, asn, Wdn, adn, Wsh, ash_, Wdh, adh, le_nc, le_hc,
               xsn_o, xsh_o, ptab_o, ftab_o, M_o):
    xsn = _dot(xp[...], Wsn[...])
    xsn_o[...] = xsn
    xsh = _dot(xf[...], Wsh[...])
    xsh_o[...] = xsh
    ls_n = _dot(xsn, asn[...])
    ld_h = _dot(_dot(xp[...], Wdh[...]), adh[...])
    ptab_o[...] = jnp.concatenate([ls_n, ld_h], axis=1)
    ld_n = _dot(_dot(xf[...], Wdn[...]), adn[...])
    ls_h = _dot(xsh, ash_[...])
    ftab_o[...] = jnp.concatenate([ld_n, ls_h], axis=1)
    raw_n = jnp.max(ls_n) + jnp.max(ld_n) + jnp.max(le_nc[...])
    raw_h = jnp.max(ls_h) + jnp.max(ld_h) + jnp.max(le_hc[...])
    M_n = jnp.where(raw_n > 0, raw_n, 0.2 * raw_n)
    M_h = jnp.where(raw_h > 0, raw_h, 0.2 * raw_h)
    M_o[...] = jnp.concatenate(
        [jnp.zeros((1, 16), _f32) + M_n, jnp.zeros((1, 16), _f32) + M_h],
        axis=0)


def _update_body(ct, ch, cn, xp, xf, xsn, xsh, Wl, bl, Wr, bias_n, bias_h,
                 lnpg, lnpb, lnfg, lnfb, ppW, ppb, pfW, pfb, xp_o, xf_o):
    i = pl.program_id(0)
    xpb = xp[pl.ds(i * 256, 256), :]
    ctb = ct[...]
    cnt = jnp.sum(ctb, axis=1, keepdims=True)
    mean = _dotH(ctb, xp[...]) / jnp.maximum(cnt, 1.0)
    sage = _dot(mean, Wl[...]) + bl[...] + _dot(xpb, Wr[...])
    chb = ch[...]
    sh = jnp.sum(chb, axis=1, keepdims=True)
    gat_p = _dotH(chb, xsh[...]) / (sh + 1e-16) + bias_h[...]
    outp = gat_p + sage
    proj = _dot(_layernorm(outp, lnpg[...], lnpb[...]), ppW[...]) + ppb[...]
    xp_o[...] = _gelu(proj + xpb)

    @pl.when(i == 0)
    def _():
        cnb = cn[...]
        sn = jnp.sum(cnb, axis=1, keepdims=True)
        outf = _dotH(cnb, xsn[...]) / (sn + 1e-16) + bias_n[...]
        projf = _dot(_layernorm(outf, lnfg[...], lnfb[...]), pfW[...]) + pfb[...]
        xf_o[...] = _gelu(projf + xf[...])


def _sprep_body(xp, xf, W1p, b1, W1f, a_o, bf_o):
    a_o[...] = _dot(xp[...], W1p[...]) + b1[...]
    bf_o[...] = _dot(xf[...], W1f[...])


def _score_body(a, bf, W2, b2, w3t, b3, o_ref):
    av = a[...]
    bfv = bf[...]
    PB = av.shape[0]
    cols = []
    for c in range(4):
        bslab = bfv[c * 32:(c + 1) * 32, :]
        h1 = _gelu(av[:, None, :] + bslab[None, :, :])
        h2 = _gelu(_dot(h1.reshape(PB * 32, 128), W2[...]) + b2[...])
        h2r = h2.reshape(PB, 32, H)
        cols.append(jnp.sum(h2r * w3t[...], axis=-1) + b3[...])
    o_ref[...] = jnp.concatenate(cols, axis=1)


def _ctemp_body(tsrc, tdst, zeros, ones, ct_o, src_v, dst_v, idx_v, ones_v,
                C_sh):
    cid = lax.axis_index("c")
    sid = lax.axis_index("s")
    pltpu.sync_copy(ones, ones_v)
    base = sid * 4096
    pltpu.sync_copy(tsrc.at[pl.ds(base, 4096)], src_v)
    pltpu.sync_copy(tdst.at[pl.ds(base, 4096)], dst_v)
    for q in range(2):
        qbase = cid * 1024 + q * 512
        pltpu.sync_copy(zeros, C_sh.at[pl.ds(sid * 65536, 65536)])
        plsc.subcore_barrier()

        @plsc.parallel_loop(0, 256, step=1, unroll=8)
        def step(j):
            off = j * 16
            s16 = src_v[pl.ds(off, 16)]
            d16 = dst_v[pl.ds(off, 16)]
            rel = d16 - qbase
            ok = (rel >= 0) & (rel < 512)
            dump = 1048576 + ((off + lax.iota(jnp.int32, 16)) & 1023)
            idx = jnp.where(ok, rel * 2048 + s16, dump)
            idx_v[pl.ds(off, 16)] = idx

        pltpu.sync_copy(ones_v, C_sh.at[idx_v], add=True)
        plsc.subcore_barrier()
        pltpu.sync_copy(
            C_sh.at[pl.ds(sid * 65536, 65536)],
            ct_o.at[pl.ds(qbase * 2048 + sid * 65536, 65536)])
        plsc.subcore_barrier()


def _edge_body(nsrc, ndst, hsrc, hdst, le_n, le_h, ptab, ftab, M, zeros,
               cn_o, ch_o, src_v, dst_v, le_v, pt_v, ft_v, M_v, idx_v, e_v,
               C_sh):
    cid = lax.axis_index("c")
    sid = lax.axis_index("s")
    pltpu.sync_copy(zeros.at[pl.ds(0, 16384)],
                    C_sh.at[pl.ds(sid * 16384, 16384)])
    pltpu.sync_copy(ptab, pt_v)
    pltpu.sync_copy(ftab, ft_v)
    pltpu.sync_copy(M.at[cid], M_v)
    base = sid * 8192

    @pl.when(cid == 0)
    def _():
        pltpu.sync_copy(nsrc.at[pl.ds(base, 8192)], src_v)
        pltpu.sync_copy(ndst.at[pl.ds(base, 8192)], dst_v)
        pltpu.sync_copy(le_n.at[pl.ds(base, 8192)], le_v)

    @pl.when(cid == 1)
    def _():
        pltpu.sync_copy(hsrc.at[pl.ds(base, 8192)], src_v)
        pltpu.sync_copy(hdst.at[pl.ds(base, 8192)], dst_v)
        pltpu.sync_copy(le_h.at[pl.ds(base, 8192)], le_v)

    plsc.subcore_barrier()

    def compute(is_near):
        @plsc.parallel_loop(0, 512, step=1, unroll=8)
        def step(j):
            off = j * 16
            s16 = src_v[pl.ds(off, 16)]
            d16 = dst_v[pl.ds(off, 16)]
            l16 = le_v[pl.ds(off, 16)]
            if is_near:
                ls = plsc.load_gather(pt_v, [s16 * 2])
                ld = plsc.load_gather(ft_v, [d16 * 2])
                idx = d16 * 2048 + s16
            else:
                ls = plsc.load_gather(ft_v, [s16 * 2 + 1])
                ld = plsc.load_gather(pt_v, [d16 * 2 + 1])
                idx = d16 * 128 + s16
            logit = ls + ld + l16
            logit = jnp.where(logit > 0, logit, logit * 0.2)
            e16 = jnp.exp(logit - M_v[...])
            e_v[pl.ds(off, 16)] = e16
            idx_v[pl.ds(off, 16)] = idx

        pltpu.sync_copy(e_v, C_sh.at[idx_v], add=True)

    @pl.when(cid == 0)
    def _():
        compute(True)

    @pl.when(cid == 1)
    def _():
        compute(False)

    plsc.subcore_barrier()

    @pl.when(cid == 0)
    def _():
        pltpu.sync_copy(C_sh.at[pl.ds(sid * 16384, 16384)],
                        cn_o.at[pl.ds(sid * 16384, 16384)])

    @pl.when(cid == 1)
    def _():
        pltpu.sync_copy(C_sh.at[pl.ds(sid * 16384, 16384)],
                        ch_o.at[pl.ds(sid * 16384, 16384)])


def _sds(shape):
    return jax.ShapeDtypeStruct(shape, _f32)


def _full_spec(shape):
    rank = len(shape)
    return pl.BlockSpec(shape, lambda i, _r=rank: (0,) * _r)


def _sc_mesh():
    return plsc.VectorSubcoreMesh(core_axis_name="c", subcore_axis_name="s",
                                  num_cores=2, num_subcores=16)


def _build_ctemp(tsrc, tdst, zeros64k, ones):
    i32 = jnp.int32
    call = pl.kernel(
        _ctemp_body,
        out_type=_sds((P * P,)),
        mesh=_sc_mesh(),
        compiler_params=pltpu.CompilerParams(needs_layout_passes=False),
        scratch_types=[
            pltpu.VMEM((4096,), i32),
            pltpu.VMEM((4096,), i32),
            pltpu.VMEM((4096,), i32),
            pltpu.VMEM((4096,), _f32),
            pltpu.VMEM_SHARED((1049600,), _f32),
        ],
    )
    return call(tsrc, tdst, zeros64k, ones)


def _build_edges(nsrc, ndst, hsrc, hdst, le_nc, le_hc, ptab, ftab, M,
                 zeros64k):
    i32 = jnp.int32
    call = pl.kernel(
        _edge_body,
        out_type=(_sds((FN * P,)), _sds((P * FN,))),
        mesh=_sc_mesh(),
        compiler_params=pltpu.CompilerParams(needs_layout_passes=False),
        scratch_types=[
            pltpu.VMEM((8192,), i32),
            pltpu.VMEM((8192,), i32),
            pltpu.VMEM((8192,), _f32),
            pltpu.VMEM((2 * P,), _f32),
            pltpu.VMEM((2 * FN,), _f32),
            pltpu.VMEM((16,), _f32),
            pltpu.VMEM((8192,), i32),
            pltpu.VMEM((8192,), _f32),
            pltpu.VMEM_SHARED((P * FN,), _f32),
        ],
    )
    return call(nsrc, ndst, hsrc, hdst, le_nc, le_hc, ptab, ftab, M, zeros64k)


def kernel(plume_x, facility_x, near_attr, hist_attr, params, near_src,
           near_dst, hist_src, hist_dst, temp_src, temp_dst):
    r1 = lambda v: v.reshape(1, -1)
    c1 = lambda v: v.reshape(-1, 1)
    i32 = jnp.int32
    nsrc = near_src.astype(i32)
    ndst = near_dst.astype(i32)
    hsrc = hist_src.astype(i32)
    hdst = hist_dst.astype(i32)
    tsrc = temp_src.astype(i32)
    tdst = temp_dst.astype(i32)

    xp, xf = pl.pallas_call(
        _init_body,
        out_shape=[_sds((P, H)), _sds((FN, H))],
    )(plume_x, facility_x, params['plume_W'], r1(params['plume_b']),
      params['fac_W'], r1(params['fac_b']))

    layers = params['layers']
    Wen = jnp.stack([L['near']['Wedge'] for L in layers])
    aen = jnp.stack([c1(L['near']['aedge']) for L in layers])
    Weh = jnp.stack([L['hist']['Wedge'] for L in layers])
    aeh = jnp.stack([c1(L['hist']['aedge']) for L in layers])

    EB = 8192
    le_near, le_hist = pl.pallas_call(
        _enc_body,
        grid=(NE // EB,),
        in_specs=[
            pl.BlockSpec((EB, 4), lambda i: (i, 0)),
            pl.BlockSpec((EB, 4), lambda i: (i, 0)),
            _full_spec((4, 32)), _full_spec((1, 32)),
            _full_spec((32, 32)), _full_spec((1, 32)),
            _full_spec((2, 32, H)), _full_spec((2, H, 1)),
            _full_spec((2, 32, H)), _full_spec((2, H, 1)),
        ],
        out_specs=[
            pl.BlockSpec((EB, 2), lambda i: (i, 0)),
            pl.BlockSpec((EB, 2), lambda i: (i, 0)),
        ],
        out_shape=[_sds((NE, 2)), _sds((NE, 2))],
    )(near_attr, hist_attr, params['enc_W1'], r1(params['enc_b1']),
      params['enc_W2'], r1(params['enc_b2']), Wen, aen, Weh, aeh)

    zeros64k = jnp.zeros((65536,), _f32)
    ones = jnp.ones((4096,), _f32)

    ct2d = _build_ctemp(tsrc, tdst, zeros64k, ones).reshape(P, P)

    for L in range(2):
        lw = layers[L]
        ln, lh = lw['near'], lw['hist']
        le_nc = lax.slice(le_near, (0, L), (NE, L + 1)).reshape(-1)
        le_hc = lax.slice(le_hist, (0, L), (NE, L + 1)).reshape(-1)
        xsn, xsh, ptab, ftab, M = pl.pallas_call(
            _prep_body,
            out_shape=[_sds((P, H)), _sds((FN, H)), _sds((P, 2)),
                       _sds((FN, 2)), _sds((2, 16))],
        )(xp, xf, ln['Wsrc'], c1(ln['asrc']), ln['Wdst'], c1(ln['adst']),
          lh['Wsrc'], c1(lh['asrc']), lh['Wdst'], c1(lh['adst']),
          le_nc.reshape(NE // 128, 128), le_hc.reshape(NE // 128, 128))

        cn_flat, ch_flat = _build_edges(
            nsrc, ndst, hsrc, hdst, le_nc, le_hc,
            ptab.reshape(-1), ftab.reshape(-1), M, zeros64k)

        sg = lw['sage']
        xp, xf = pl.pallas_call(
            _update_body,
            grid=(P // 256,),
            in_specs=[
                pl.BlockSpec((256, P), lambda i: (i, 0)),
                pl.BlockSpec((256, FN), lambda i: (i, 0)),
                _full_spec((FN, P)),
                _full_spec((P, H)), _full_spec((FN, H)),
                _full_spec((P, H)), _full_spec((FN, H)),
                _full_spec((H, H)), _full_spec((1, H)), _full_spec((H, H)),
                _full_spec((1, H)), _full_spec((1, H)),
                _full_spec((1, H)), _full_spec((1, H)),
                _full_spec((1, H)), _full_spec((1, H)),
                _full_spec((H, H)), _full_spec((1, H)),
                _full_spec((H, H)), _full_spec((1, H)),
            ],
            out_specs=[
                pl.BlockSpec((256, H), lambda i: (i, 0)),
                pl.BlockSpec((FN, H), lambda i: (0, 0)),
            ],
            out_shape=[_sds((P, H)), _sds((FN, H))],
        )(ct2d, ch_flat.reshape(P, FN), cn_flat.reshape(FN, P), xp, xf,
          xsn, xsh, sg['Wl'], r1(sg['bl']), sg['Wr'], r1(ln['bias']),
          r1(lh['bias']), r1(lw['ln_p_g']), r1(lw['ln_p_b']),
          r1(lw['ln_f_g']), r1(lw['ln_f_b']), lw['proj_p_W'],
          r1(lw['proj_p_b']), lw['proj_f_W'], r1(lw['proj_f_b']))

    a, bf = pl.pallas_call(
        _sprep_body,
        out_shape=[_sds((P, 128)), _sds((FN, 128))],
    )(xp, xf, params['s_W1'][:H], r1(params['s_b1']), params['s_W1'][H:])

    PB = 128
    out = pl.pallas_call(
        _score_body,
        grid=(P // PB,),
        in_specs=[
            pl.BlockSpec((PB, 128), lambda i: (i, 0)),
            _full_spec((FN, 128)),
            _full_spec((128, H)), _full_spec((1, H)),
            _full_spec((1, 1, H)), _full_spec((1, 1)),
        ],
        out_specs=pl.BlockSpec((PB, FN), lambda i: (i, 0)),
        out_shape=_sds((P, FN)),
    )(a, bf, params['s_W2'], r1(params['s_b2']),
      params['s_W3'].reshape(1, 1, H), params['s_b3'].reshape(1, 1))

    return out


# speedup vs baseline: 1.2516x; 1.0026x over previous
import math

import jax
import jax.numpy as jnp
from jax import lax
from jax.experimental import pallas as pl
from jax.experimental.pallas import tpu as pltpu
from jax.experimental.pallas import tpu_sc as plsc

P = 2048
FN = 128
H = 64
NE = 131072
NT = 65536
LN_DECAY = math.log(0.9)
INV_SQRT2 = 1.0 / math.sqrt(2.0)

_f32 = jnp.float32


def _gelu(x):
    return 0.5 * x * (1.0 + lax.erf(x * INV_SQRT2))


def _layernorm(x, g, b):
    m = jnp.mean(x, axis=-1, keepdims=True)
    v = jnp.mean((x - m) ** 2, axis=-1, keepdims=True)
    return (x - m) * lax.rsqrt(v + 1e-5) * g + b


def _dot(a, b):
    return jnp.dot(a, b, preferred_element_type=jnp.float32)


def _dotH(a, b):
    return jnp.dot(a, b, preferred_element_type=jnp.float32,
                   precision=lax.Precision.HIGHEST)


def _enc_body(na, ha, W1, b1, W2, b2, Wen, aen, Weh, aeh, px, fx, pW, pb,
              fW, fb, le_n_o, le_h_o, xp_o, xf_o):
    @pl.when(pl.program_id(0) == 0)
    def _():
        xp_o[...] = _gelu(_dot(px[...], pW[...]) + pb[...])
        xf_o[...] = _gelu(_dot(fx[...], fW[...]) + fb[...])

    def enc(attr):
        h = _gelu(_dot(attr, W1[...]) + b1[...])
        h = _dot(h, W2[...]) + b2[...]
        return h * jnp.exp(attr[:, 2:3] * LN_DECAY)

    e_n = enc(na[...])
    e_h = enc(ha[...])
    le_n_o[...] = jnp.concatenate(
        [_dot(e_n, _dot(Wen[L], aen[L])) for L in range(2)], axis=1)
    le_h_o[...] = jnp.concatenate(
        [_dot(e_h, _dot(Weh[L], aeh[L])) for L in range(2)], axis=1)


def _prep_body(xp, xf, Wsn, asn, Wdn, adn, Wsh, ash_, Wdh, adh, le_nc, le_hc,
               xsn_o, xsh_o, ptab_o, ftab_o, M_o):
    xsn = _dot(xp[...], Wsn[...])
    xsn_o[...] = xsn
    xsh = _dot(xf[...], Wsh[...])
    xsh_o[...] = xsh
    ls_n = _dot(xsn, asn[...])
    ld_h = _dot(_dot(xp[...], Wdh[...]), adh[...])
    ptab_o[...] = jnp.concatenate([ls_n, ld_h], axis=1)
    ld_n = _dot(_dot(xf[...], Wdn[...]), adn[...])
    ls_h = _dot(xsh, ash_[...])
    ftab_o[...] = jnp.concatenate([ld_n, ls_h], axis=1)
    raw_n = jnp.max(ls_n) + jnp.max(ld_n) + jnp.max(le_nc[...])
    raw_h = jnp.max(ls_h) + jnp.max(ld_h) + jnp.max(le_hc[...])
    M_n = jnp.where(raw_n > 0, raw_n, 0.2 * raw_n)
    M_h = jnp.where(raw_h > 0, raw_h, 0.2 * raw_h)
    M_o[...] = jnp.concatenate(
        [jnp.zeros((1, 16), _f32) + M_n, jnp.zeros((1, 16), _f32) + M_h],
        axis=0)


def _update_core(ct, ch, cn, xp, xf, xsn, xsh, Wl, bl, Wr, bias_n, bias_h,
                 lnpg, lnpb, lnfg, lnfb, ppW, ppb, pfW, pfb, xp_o, xf_o):
    i = pl.program_id(0)
    xpb = xp[pl.ds(i * 256, 256), :]
    ctb = ct[...]
    cnt = jnp.sum(ctb, axis=1, keepdims=True)
    mean = _dotH(ctb, xp[...]) / jnp.maximum(cnt, 1.0)
    sage = _dot(mean, Wl[...]) + bl[...] + _dot(xpb, Wr[...])
    chb = ch[...]
    sh = jnp.sum(chb, axis=1, keepdims=True)
    gat_p = _dotH(chb, xsh[...]) / (sh + 1e-16) + bias_h[...]
    outp = gat_p + sage
    proj = _dot(_layernorm(outp, lnpg[...], lnpb[...]), ppW[...]) + ppb[...]
    xp_o[...] = _gelu(proj + xpb)

    @pl.when(i == 0)
    def _():
        cnb = cn[...]
        sn = jnp.sum(cnb, axis=1, keepdims=True)
        outf = _dotH(cnb, xsn[...]) / (sn + 1e-16) + bias_n[...]
        projf = _dot(_layernorm(outf, lnfg[...], lnfb[...]), pfW[...]) + pfb[...]
        xf_o[...] = _gelu(projf + xf[...])


def _update_body(*refs):
    _update_core(*refs)


def _update_sprep_body(ct, ch, cn, xp, xf, xsn, xsh, Wl, bl, Wr, bias_n,
                       bias_h, lnpg, lnpb, lnfg, lnfb, ppW, ppb, pfW, pfb,
                       W1p, sb1, W1f, xp_o, xf_o, a_o, bf_o):
    _update_core(ct, ch, cn, xp, xf, xsn, xsh, Wl, bl, Wr, bias_n, bias_h,
                 lnpg, lnpb, lnfg, lnfb, ppW, ppb, pfW, pfb, xp_o, xf_o)
    a_o[...] = _dot(xp_o[...], W1p[...]) + sb1[...]

    @pl.when(pl.program_id(0) == 0)
    def _():
        bf_o[...] = _dot(xf_o[...], W1f[...])


def _sprep_body(xp, xf, W1p, b1, W1f, a_o, bf_o):
    a_o[...] = _dot(xp[...], W1p[...]) + b1[...]
    bf_o[...] = _dot(xf[...], W1f[...])


def _score_body(a, bf, W2, b2, w3t, b3, o_ref):
    av = a[...]
    bfv = bf[...]
    PB = av.shape[0]
    cols = []
    for c in range(4):
        bslab = bfv[c * 32:(c + 1) * 32, :]
        h1 = _gelu(av[:, None, :] + bslab[None, :, :])
        h2 = _gelu(_dot(h1.reshape(PB * 32, 128), W2[...]) + b2[...])
        h2r = h2.reshape(PB, 32, H)
        cols.append(jnp.sum(h2r * w3t[...], axis=-1) + b3[...])
    o_ref[...] = jnp.concatenate(cols, axis=1)


def _ctemp_body(tsrc, tdst, zeros, ones, ct_o, src_v, dst_v, idx_v, ones_v,
                C_sh):
    cid = lax.axis_index("c")
    sid = lax.axis_index("s")
    pltpu.sync_copy(ones, ones_v)
    base = sid * 4096
    pltpu.sync_copy(tsrc.at[pl.ds(base, 4096)], src_v)
    pltpu.sync_copy(tdst.at[pl.ds(base, 4096)], dst_v)
    for q in range(2):
        qbase = cid * 1024 + q * 512
        pltpu.sync_copy(zeros, C_sh.at[pl.ds(sid * 65536, 65536)])
        plsc.subcore_barrier()

        @plsc.parallel_loop(0, 256, step=1, unroll=8)
        def step(j):
            off = j * 16
            s16 = src_v[pl.ds(off, 16)]
            d16 = dst_v[pl.ds(off, 16)]
            rel = d16 - qbase
            ok = (rel >= 0) & (rel < 512)
            dump = 1048576 + ((off + lax.iota(jnp.int32, 16)) & 1023)
            idx = jnp.where(ok, rel * 2048 + s16, dump)
            idx_v[pl.ds(off, 16)] = idx

        pltpu.sync_copy(ones_v, C_sh.at[idx_v], add=True)
        plsc.subcore_barrier()
        pltpu.sync_copy(
            C_sh.at[pl.ds(sid * 65536, 65536)],
            ct_o.at[pl.ds(qbase * 2048 + sid * 65536, 65536)])
        plsc.subcore_barrier()


def _edge_body(nsrc, ndst, hsrc, hdst, le_n, le_h, ptab, ftab, M, zeros,
               cn_o, ch_o, src_v, dst_v, le_v, pt_v, ft_v, M_v, idx_v, e_v,
               C_sh):
    cid = lax.axis_index("c")
    sid = lax.axis_index("s")
    pltpu.sync_copy(zeros.at[pl.ds(0, 16384)],
                    C_sh.at[pl.ds(sid * 16384, 16384)])
    pltpu.sync_copy(ptab, pt_v)
    pltpu.sync_copy(ftab, ft_v)
    pltpu.sync_copy(M.at[cid], M_v)
    base = sid * 8192

    @pl.when(cid == 0)
    def _():
        pltpu.sync_copy(nsrc.at[pl.ds(base, 8192)], src_v)
        pltpu.sync_copy(ndst.at[pl.ds(base, 8192)], dst_v)
        pltpu.sync_copy(le_n.at[pl.ds(base, 8192)], le_v)

    @pl.when(cid == 1)
    def _():
        pltpu.sync_copy(hsrc.at[pl.ds(base, 8192)], src_v)
        pltpu.sync_copy(hdst.at[pl.ds(base, 8192)], dst_v)
        pltpu.sync_copy(le_h.at[pl.ds(base, 8192)], le_v)

    plsc.subcore_barrier()

    def compute(is_near):
        @plsc.parallel_loop(0, 512, step=1, unroll=8)
        def step(j):
            off = j * 16
            s16 = src_v[pl.ds(off, 16)]
            d16 = dst_v[pl.ds(off, 16)]
            l16 = le_v[pl.ds(off, 16)]
            if is_near:
                ls = plsc.load_gather(pt_v, [s16 * 2])
                ld = plsc.load_gather(ft_v, [d16 * 2])
                idx = d16 * 2048 + s16
            else:
                ls = plsc.load_gather(ft_v, [s16 * 2 + 1])
                ld = plsc.load_gather(pt_v, [d16 * 2 + 1])
                idx = d16 * 128 + s16
            logit = ls + ld + l16
            logit = jnp.where(logit > 0, logit, logit * 0.2)
            e16 = jnp.exp(logit - M_v[...])
            e_v[pl.ds(off, 16)] = e16
            idx_v[pl.ds(off, 16)] = idx

        pltpu.sync_copy(e_v, C_sh.at[idx_v], add=True)

    @pl.when(cid == 0)
    def _():
        compute(True)

    @pl.when(cid == 1)
    def _():
        compute(False)

    plsc.subcore_barrier()

    @pl.when(cid == 0)
    def _():
        pltpu.sync_copy(C_sh.at[pl.ds(sid * 16384, 16384)],
                        cn_o.at[pl.ds(sid * 16384, 16384)])

    @pl.when(cid == 1)
    def _():
        pltpu.sync_copy(C_sh.at[pl.ds(sid * 16384, 16384)],
                        ch_o.at[pl.ds(sid * 16384, 16384)])


def _sds(shape):
    return jax.ShapeDtypeStruct(shape, _f32)


def _full_spec(shape):
    rank = len(shape)
    return pl.BlockSpec(shape, lambda i, _r=rank: (0,) * _r)


def _sc_mesh():
    return plsc.VectorSubcoreMesh(core_axis_name="c", subcore_axis_name="s",
                                  num_cores=2, num_subcores=16)


def _build_ctemp(tsrc, tdst, zeros64k, ones):
    i32 = jnp.int32
    call = pl.kernel(
        _ctemp_body,
        out_type=_sds((P * P,)),
        mesh=_sc_mesh(),
        compiler_params=pltpu.CompilerParams(needs_layout_passes=False),
        scratch_types=[
            pltpu.VMEM((4096,), i32),
            pltpu.VMEM((4096,), i32),
            pltpu.VMEM((4096,), i32),
            pltpu.VMEM((4096,), _f32),
            pltpu.VMEM_SHARED((1049600,), _f32),
        ],
    )
    return call(tsrc, tdst, zeros64k, ones)


def _build_edges(nsrc, ndst, hsrc, hdst, le_nc, le_hc, ptab, ftab, M,
                 zeros64k):
    i32 = jnp.int32
    call = pl.kernel(
        _edge_body,
        out_type=(_sds((FN * P,)), _sds((P * FN,))),
        mesh=_sc_mesh(),
        compiler_params=pltpu.CompilerParams(needs_layout_passes=False),
        scratch_types=[
            pltpu.VMEM((8192,), i32),
            pltpu.VMEM((8192,), i32),
            pltpu.VMEM((8192,), _f32),
            pltpu.VMEM((2 * P,), _f32),
            pltpu.VMEM((2 * FN,), _f32),
            pltpu.VMEM((16,), _f32),
            pltpu.VMEM((8192,), i32),
            pltpu.VMEM((8192,), _f32),
            pltpu.VMEM_SHARED((P * FN,), _f32),
        ],
    )
    return call(nsrc, ndst, hsrc, hdst, le_nc, le_hc, ptab, ftab, M, zeros64k)


def kernel(plume_x, facility_x, near_attr, hist_attr, params, near_src,
           near_dst, hist_src, hist_dst, temp_src, temp_dst):
    r1 = lambda v: v.reshape(1, -1)
    c1 = lambda v: v.reshape(-1, 1)
    i32 = jnp.int32
    nsrc = near_src.astype(i32)
    ndst = near_dst.astype(i32)
    hsrc = hist_src.astype(i32)
    hdst = hist_dst.astype(i32)
    tsrc = temp_src.astype(i32)
    tdst = temp_dst.astype(i32)

    layers = params['layers']
    Wen = jnp.stack([L['near']['Wedge'] for L in layers])
    aen = jnp.stack([c1(L['near']['aedge']) for L in layers])
    Weh = jnp.stack([L['hist']['Wedge'] for L in layers])
    aeh = jnp.stack([c1(L['hist']['aedge']) for L in layers])

    EB = 8192
    le_near, le_hist, xp, xf = pl.pallas_call(
        _enc_body,
        grid=(NE // EB,),
        in_specs=[
            pl.BlockSpec((EB, 4), lambda i: (i, 0)),
            pl.BlockSpec((EB, 4), lambda i: (i, 0)),
            _full_spec((4, 32)), _full_spec((1, 32)),
            _full_spec((32, 32)), _full_spec((1, 32)),
            _full_spec((2, 32, H)), _full_spec((2, H, 1)),
            _full_spec((2, 32, H)), _full_spec((2, H, 1)),
            _full_spec((P, 8)), _full_spec((FN, 6)),
            _full_spec((8, H)), _full_spec((1, H)),
            _full_spec((6, H)), _full_spec((1, H)),
        ],
        out_specs=[
            pl.BlockSpec((EB, 2), lambda i: (i, 0)),
            pl.BlockSpec((EB, 2), lambda i: (i, 0)),
            pl.BlockSpec((P, H), lambda i: (0, 0)),
            pl.BlockSpec((FN, H), lambda i: (0, 0)),
        ],
        out_shape=[_sds((NE, 2)), _sds((NE, 2)), _sds((P, H)),
                   _sds((FN, H))],
    )(near_attr, hist_attr, params['enc_W1'], r1(params['enc_b1']),
      params['enc_W2'], r1(params['enc_b2']), Wen, aen, Weh, aeh,
      plume_x, facility_x, params['plume_W'], r1(params['plume_b']),
      params['fac_W'], r1(params['fac_b']))

    zeros64k = jnp.zeros((65536,), _f32)
    ones = jnp.ones((4096,), _f32)

    ct2d = _build_ctemp(tsrc, tdst, zeros64k, ones).reshape(P, P)

    for L in range(2):
        lw = layers[L]
        ln, lh = lw['near'], lw['hist']
        le_nc = lax.slice(le_near, (0, L), (NE, L + 1)).reshape(-1)
        le_hc = lax.slice(le_hist, (0, L), (NE, L + 1)).reshape(-1)
        xsn, xsh, ptab, ftab, M = pl.pallas_call(
            _prep_body,
            out_shape=[_sds((P, H)), _sds((FN, H)), _sds((P, 2)),
                       _sds((FN, 2)), _sds((2, 16))],
        )(xp, xf, ln['Wsrc'], c1(ln['asrc']), ln['Wdst'], c1(ln['adst']),
          lh['Wsrc'], c1(lh['asrc']), lh['Wdst'], c1(lh['adst']),
          le_nc.reshape(NE // 128, 128), le_hc.reshape(NE // 128, 128))

        cn_flat, ch_flat = _build_edges(
            nsrc, ndst, hsrc, hdst, le_nc, le_hc,
            ptab.reshape(-1), ftab.reshape(-1), M, zeros64k)

        sg = lw['sage']
        base_specs = [
            pl.BlockSpec((256, P), lambda i: (i, 0)),
            pl.BlockSpec((256, FN), lambda i: (i, 0)),
            _full_spec((FN, P)),
            _full_spec((P, H)), _full_spec((FN, H)),
            _full_spec((P, H)), _full_spec((FN, H)),
            _full_spec((H, H)), _full_spec((1, H)), _full_spec((H, H)),
            _full_spec((1, H)), _full_spec((1, H)),
            _full_spec((1, H)), _full_spec((1, H)),
            _full_spec((1, H)), _full_spec((1, H)),
            _full_spec((H, H)), _full_spec((1, H)),
            _full_spec((H, H)), _full_spec((1, H)),
        ]
        base_args = (ct2d, ch_flat.reshape(P, FN), cn_flat.reshape(FN, P),
                     xp, xf, xsn, xsh, sg['Wl'], r1(sg['bl']), sg['Wr'],
                     r1(ln['bias']), r1(lh['bias']), r1(lw['ln_p_g']),
                     r1(lw['ln_p_b']), r1(lw['ln_f_g']), r1(lw['ln_f_b']),
                     lw['proj_p_W'], r1(lw['proj_p_b']), lw['proj_f_W'],
                     r1(lw['proj_f_b']))
        base_out_specs = [
            pl.BlockSpec((256, H), lambda i: (i, 0)),
            pl.BlockSpec((FN, H), lambda i: (0, 0)),
        ]
        if L == 0:
            xp, xf = pl.pallas_call(
                _update_body,
                grid=(P // 256,),
                in_specs=base_specs,
                out_specs=base_out_specs,
                out_shape=[_sds((P, H)), _sds((FN, H))],
            )(*base_args)
        else:
            xp, xf, a, bf = pl.pallas_call(
                _update_sprep_body,
                grid=(P // 256,),
                in_specs=base_specs + [
                    _full_spec((H, 128)), _full_spec((1, 128)),
                    _full_spec((H, 128)),
                ],
                out_specs=base_out_specs + [
                    pl.BlockSpec((256, 128), lambda i: (i, 0)),
                    pl.BlockSpec((FN, 128), lambda i: (0, 0)),
                ],
                out_shape=[_sds((P, H)), _sds((FN, H)), _sds((P, 128)),
                           _sds((FN, 128))],
            )(*(base_args + (params['s_W1'][:H], r1(params['s_b1']),
                             params['s_W1'][H:])))

    PB = 128
    out = pl.pallas_call(
        _score_body,
        grid=(P // PB,),
        in_specs=[
            pl.BlockSpec((PB, 128), lambda i: (i, 0)),
            _full_spec((FN, 128)),
            _full_spec((128, H)), _full_spec((1, H)),
            _full_spec((1, 1, H)), _full_spec((1, 1)),
        ],
        out_specs=pl.BlockSpec((PB, FN), lambda i: (i, 0)),
        out_shape=_sds((P, FN)),
    )(a, bf, params['s_W2'], r1(params['s_b2']),
      params['s_W3'].reshape(1, 1, H), params['s_b3'].reshape(1, 1))

    return out
